# baseline (speedup 1.0000x reference)
import functools

import jax
import jax.numpy as jnp
from jax import lax
from jax.experimental import pallas as pl
from jax.experimental.pallas import tpu as pltpu
from jax.experimental.pallas import tpu_sc as plsc

N = 10000
E = 320000
IN_DIM = 128
H1 = 128
H2 = 128
OUT = 64

NC = 2
NS = 16
NW = NC * NS
K = 64
NBUF = 4
EP = 327680
TCH = EP // K
DCH = TCH // NW
NACC = 10240
RPT = NACC // NS
DEG_D = 16
RB = 1024
GRID = NACC // RB

_mesh = lambda: plsc.VectorSubcoreMesh(
    core_axis_name="c", subcore_axis_name="s", num_cores=NC)


def _make_sc_agg(D, KA, PHA):
    tch = EP // KA
    ch = tch // NW
    nph = ch // PHA
    assert ch * NW == tch and nph * PHA == ch and PHA % NBUF == 0

    @functools.partial(
        pl.kernel,
        out_type=jax.ShapeDtypeStruct((NC, NACC, D), jnp.float32),
        mesh=_mesh(),
        scratch_types=[
            pltpu.VMEM((PHA, KA), jnp.int32),
            pltpu.VMEM((PHA, KA), jnp.int32),
            [pltpu.VMEM((KA, D), jnp.float32) for _ in range(NBUF)],
            pltpu.VMEM_SHARED((NACC, D), jnp.float32),
            [pltpu.SemaphoreType.DMA for _ in range(NBUF)],
        ],
        compiler_params=pltpu.CompilerParams(use_tc_tiling_on_sc=False),
    )
    def agg(h_hbm, src_hbm, dst_hbm, zeros_hbm, out_hbm,
            sidx, didx, rows, acc, gsems):
        c = lax.axis_index("c")
        s = lax.axis_index("s")
        r0 = s * RPT
        start = (c * NS + s) * ch
        pltpu.sync_copy(zeros_hbm, acc.at[pl.ds(r0, RPT)])
        plsc.subcore_barrier()

        def one(ci, b):
            pltpu.make_async_copy(h_hbm.at[sidx.at[ci]], rows[b],
                                  gsems[b]).wait()
            pltpu.sync_copy(rows[b], acc.at[didx.at[ci]], add=True)

            @pl.when(ci + NBUF < PHA)
            def _():
                pltpu.async_copy(h_hbm.at[sidx.at[ci + NBUF]], rows[b],
                                 gsems[b])

        def group(tt, carry):
            for b in range(NBUF):
                one(NBUF * tt + b, b)
            return carry

        for ph in range(nph):
            pbase = start + ph * PHA
            pltpu.sync_copy(src_hbm.at[pl.ds(pbase, PHA)], sidx)
            pltpu.sync_copy(dst_hbm.at[pl.ds(pbase, PHA)], didx)
            for b in range(NBUF):
                pltpu.async_copy(h_hbm.at[sidx.at[b]], rows[b], gsems[b])
            lax.fori_loop(0, PHA // NBUF, group, 0)

        plsc.subcore_barrier()
        pltpu.sync_copy(acc.at[pl.ds(r0, RPT)], out_hbm.at[c, pl.ds(r0, RPT)])

    return agg


K128 = 64
K64 = 128
_sc_agg128 = _make_sc_agg(128, K128, 40)
_sc_agg64 = _make_sc_agg(OUT, K64, 40)


@functools.partial(
    pl.kernel,
    out_type=jax.ShapeDtypeStruct((NC, NACC, DEG_D), jnp.float32),
    mesh=_mesh(),
    scratch_types=[
        pltpu.VMEM((DCH, K), jnp.int32),
        pltpu.VMEM((K, DEG_D), jnp.float32),
        pltpu.VMEM_SHARED((NACC, DEG_D), jnp.float32),
    ],
    compiler_params=pltpu.CompilerParams(use_tc_tiling_on_sc=False),
)
def _sc_deg(dst_hbm, ones_hbm, zeros_hbm, out_hbm, didx, ones_v, acc):
    c = lax.axis_index("c")
    s = lax.axis_index("s")
    r0 = s * RPT
    wid = c * NS + s
    pltpu.sync_copy(dst_hbm.at[pl.ds(wid * DCH, DCH)], didx)
    pltpu.sync_copy(ones_hbm, ones_v)
    pltpu.sync_copy(zeros_hbm, acc.at[pl.ds(r0, RPT)])
    plsc.subcore_barrier()

    def chunk(i, carry):
        pltpu.sync_copy(ones_v, acc.at[didx.at[i]], add=True)
        return carry

    lax.fori_loop(0, DCH, chunk, 0)
    plsc.subcore_barrier()
    pltpu.sync_copy(acc.at[pl.ds(r0, RPT)], out_hbm.at[c, pl.ds(r0, RPT)])


def _tc_matmul(x_p, W1):
    def body(x_ref, w_ref, o_ref):
        o_ref[...] = jnp.dot(x_ref[...], w_ref[...],
                             preferred_element_type=jnp.float32)

    return pl.pallas_call(
        body,
        grid=(GRID,),
        in_specs=[
            pl.BlockSpec((RB, IN_DIM), lambda i: (i, 0)),
            pl.BlockSpec((IN_DIM, H1), lambda i: (0, 0)),
        ],
        out_specs=pl.BlockSpec((RB, H1), lambda i: (i, 0)),
        out_shape=jax.ShapeDtypeStruct((NACC, H1), jnp.float32),
    )(x_p, W1)


def _tc_scale(t1, p0, p1):
    def body(t_ref, p0_ref, p1_ref, g_ref, dv_ref):
        deg = p0_ref[...] + p1_ref[...] + 1.0
        dv = lax.rsqrt(deg)
        dv_ref[...] = dv
        g_ref[...] = t_ref[...] * dv[:, 0:1]

    return pl.pallas_call(
        body,
        grid=(GRID,),
        in_specs=[
            pl.BlockSpec((RB, H1), lambda i: (i, 0)),
            pl.BlockSpec((RB, DEG_D), lambda i: (i, 0)),
            pl.BlockSpec((RB, DEG_D), lambda i: (i, 0)),
        ],
        out_specs=[
            pl.BlockSpec((RB, H1), lambda i: (i, 0)),
            pl.BlockSpec((RB, DEG_D), lambda i: (i, 0)),
        ],
        out_shape=[
            jax.ShapeDtypeStruct((NACC, H1), jnp.float32),
            jax.ShapeDtypeStruct((NACC, DEG_D), jnp.float32),
        ],
    )(t1, p0, p1)


def _tc_mid(sa, sb, g, dv, b, W, d_in, d_out):
    def body(sa_ref, sb_ref, g_ref, dv_ref, b_ref, w_ref, o_ref):
        dvc = dv_ref[...][:, 0:1]
        z = jnp.maximum(
            dvc * (sa_ref[...] + sb_ref[...] + g_ref[...]) + b_ref[...], 0.0)
        o_ref[...] = jnp.dot(z, w_ref[...],
                             preferred_element_type=jnp.float32) * dvc

    return pl.pallas_call(
        body,
        grid=(GRID,),
        in_specs=[
            pl.BlockSpec((RB, d_in), lambda i: (i, 0)),
            pl.BlockSpec((RB, d_in), lambda i: (i, 0)),
            pl.BlockSpec((RB, d_in), lambda i: (i, 0)),
            pl.BlockSpec((RB, DEG_D), lambda i: (i, 0)),
            pl.BlockSpec((1, d_in), lambda i: (0, 0)),
            pl.BlockSpec((d_in, d_out), lambda i: (0, 0)),
        ],
        out_specs=pl.BlockSpec((RB, d_out), lambda i: (i, 0)),
        out_shape=jax.ShapeDtypeStruct((NACC, d_out), jnp.float32),
    )(sa, sb, g, dv, b, W)


def _tc_final(sa, sb, g, dv, b):
    rb = N // 10

    def body(sa_ref, sb_ref, g_ref, dv_ref, b_ref, o_ref):
        dvc = dv_ref[...][:, 0:1]
        z = dvc * (sa_ref[...] + sb_ref[...] + g_ref[...]) + b_ref[...]
        m = jnp.max(z, axis=1, keepdims=True)
        e = jnp.exp(z - m)
        o_ref[...] = z - m - jnp.log(jnp.sum(e, axis=1, keepdims=True))

    return pl.pallas_call(
        body,
        grid=(10,),
        in_specs=[
            pl.BlockSpec((rb, OUT), lambda i: (i, 0)),
            pl.BlockSpec((rb, OUT), lambda i: (i, 0)),
            pl.BlockSpec((rb, OUT), lambda i: (i, 0)),
            pl.BlockSpec((rb, DEG_D), lambda i: (i, 0)),
            pl.BlockSpec((1, OUT), lambda i: (0, 0)),
        ],
        out_specs=pl.BlockSpec((rb, OUT), lambda i: (i, 0)),
        out_shape=jax.ShapeDtypeStruct((N, OUT), jnp.float32),
    )(sa, sb, g, dv, b)


def kernel(x, edge_index, W1, b1, W2, b2, W3, b3):
    src = edge_index[0]
    dst = edge_index[1]
    pad = EP - E
    pad_idx = (jnp.arange(pad, dtype=jnp.int32) % (NACC - N - 16)) + N + 16
    src_f = jnp.concatenate([src, pad_idx])
    dst_f = jnp.concatenate([dst, pad_idx])
    src_p = src_f.reshape(TCH, K)
    dst_p = dst_f.reshape(TCH, K)
    src_w = src_f.reshape(EP // K64, K64)
    dst_w = dst_f.reshape(EP // K64, K64)
    ones16 = jnp.ones((K, DEG_D), jnp.float32)
    z16 = jnp.zeros((RPT, DEG_D), jnp.float32)
    z128 = jnp.zeros((RPT, 128), jnp.float32)
    z64 = jnp.zeros((RPT, OUT), jnp.float32)
    x_p = jnp.pad(x, ((0, NACC - N), (0, 0)))

    degP = _sc_deg(dst_p, ones16, z16)
    t1 = _tc_matmul(x_p, W1)
    g1, dv = _tc_scale(t1, degP[0], degP[1])
    S1 = _sc_agg128(g1, src_p, dst_p, z128)
    g2 = _tc_mid(S1[0], S1[1], g1, dv, b1.reshape(1, -1), W2, H1, H2)
    S2 = _sc_agg128(g2, src_p, dst_p, z128)
    g3 = _tc_mid(S2[0], S2[1], g2, dv, b2.reshape(1, -1), W3, H2, OUT)
    S3 = _sc_agg64(g3, src_w, dst_w, z64)
    return _tc_final(S3[0], S3[1], g3, dv, b3.reshape(1, -1))


# speedup vs baseline: 1.0403x; 1.0403x over previous
import functools

import jax
import jax.numpy as jnp
from jax import lax
from jax.experimental import pallas as pl
from jax.experimental.pallas import tpu as pltpu
from jax.experimental.pallas import tpu_sc as plsc

N = 10000
E = 320000
IN_DIM = 128
H1 = 128
H2 = 128
OUT = 64

NC = 2
NS = 16
NW = NC * NS
K = 64
NBUF = 4
EP = 327680
TCH = EP // K
DCH = TCH // NW
NACC = 10240
RPT = NACC // NS
DEG_D = 16
RB = 1024
GRID = NACC // RB

_mesh = lambda: plsc.VectorSubcoreMesh(
    core_axis_name="c", subcore_axis_name="s", num_cores=NC)


def _make_sc_agg(D, KA, PHA):
    tch = EP // KA
    ch = tch // NW
    nph = ch // PHA
    assert ch * NW == tch and nph * PHA == ch and PHA % NBUF == 0

    @functools.partial(
        pl.kernel,
        out_type=jax.ShapeDtypeStruct((NC, NACC, D), jnp.float32),
        mesh=_mesh(),
        scratch_types=[
            pltpu.VMEM((PHA, KA), jnp.int32),
            pltpu.VMEM((PHA, KA), jnp.int32),
            [pltpu.VMEM((KA, D), jnp.float32) for _ in range(NBUF)],
            pltpu.VMEM_SHARED((NACC, D), jnp.float32),
            [pltpu.SemaphoreType.DMA for _ in range(NBUF)],
        ],
        compiler_params=pltpu.CompilerParams(use_tc_tiling_on_sc=False),
    )
    def agg(h_hbm, src_hbm, dst_hbm, zeros_hbm, out_hbm,
            sidx, didx, rows, acc, gsems):
        c = lax.axis_index("c")
        s = lax.axis_index("s")
        r0 = s * RPT
        start = (c * NS + s) * ch
        pltpu.sync_copy(zeros_hbm, acc.at[pl.ds(r0, RPT)])
        plsc.subcore_barrier()

        def one(ci, b):
            pltpu.make_async_copy(h_hbm.at[sidx.at[ci]], rows[b],
                                  gsems[b]).wait()
            pltpu.sync_copy(rows[b], acc.at[didx.at[ci]], add=True)

            @pl.when(ci + NBUF < PHA)
            def _():
                pltpu.async_copy(h_hbm.at[sidx.at[ci + NBUF]], rows[b],
                                 gsems[b])

        def group(tt, carry):
            for b in range(NBUF):
                one(NBUF * tt + b, b)
            return carry

        for ph in range(nph):
            pbase = start + ph * PHA
            pltpu.sync_copy(src_hbm.at[pl.ds(pbase, PHA)], sidx)
            pltpu.sync_copy(dst_hbm.at[pl.ds(pbase, PHA)], didx)
            for b in range(NBUF):
                pltpu.async_copy(h_hbm.at[sidx.at[b]], rows[b], gsems[b])
            lax.fori_loop(0, PHA // NBUF, group, 0)

        plsc.subcore_barrier()
        pltpu.sync_copy(acc.at[pl.ds(r0, RPT)], out_hbm.at[c, pl.ds(r0, RPT)])

    return agg


K128 = 64
K64 = 128
_sc_agg128 = _make_sc_agg(128, K128, 80)
_sc_agg64 = _make_sc_agg(OUT, K64, 80)


@functools.partial(
    pl.kernel,
    out_type=jax.ShapeDtypeStruct((NC, NACC, DEG_D), jnp.float32),
    mesh=_mesh(),
    scratch_types=[
        pltpu.VMEM((DCH, K), jnp.int32),
        pltpu.VMEM((K, DEG_D), jnp.float32),
        pltpu.VMEM_SHARED((NACC, DEG_D), jnp.float32),
    ],
    compiler_params=pltpu.CompilerParams(use_tc_tiling_on_sc=False),
)
def _sc_deg(dst_hbm, ones_hbm, zeros_hbm, out_hbm, didx, ones_v, acc):
    c = lax.axis_index("c")
    s = lax.axis_index("s")
    r0 = s * RPT
    wid = c * NS + s
    pltpu.sync_copy(dst_hbm.at[pl.ds(wid * DCH, DCH)], didx)
    pltpu.sync_copy(ones_hbm, ones_v)
    pltpu.sync_copy(zeros_hbm, acc.at[pl.ds(r0, RPT)])
    plsc.subcore_barrier()

    def chunk(i, carry):
        pltpu.sync_copy(ones_v, acc.at[didx.at[i]], add=True)
        return carry

    lax.fori_loop(0, DCH, chunk, 0)
    plsc.subcore_barrier()
    pltpu.sync_copy(acc.at[pl.ds(r0, RPT)], out_hbm.at[c, pl.ds(r0, RPT)])


def _tc_matmul(x_p, W1):
    def body(x_ref, w_ref, o_ref):
        o_ref[...] = jnp.dot(x_ref[...], w_ref[...],
                             preferred_element_type=jnp.float32)

    return pl.pallas_call(
        body,
        grid=(GRID,),
        in_specs=[
            pl.BlockSpec((RB, IN_DIM), lambda i: (i, 0)),
            pl.BlockSpec((IN_DIM, H1), lambda i: (0, 0)),
        ],
        out_specs=pl.BlockSpec((RB, H1), lambda i: (i, 0)),
        out_shape=jax.ShapeDtypeStruct((NACC, H1), jnp.float32),
    )(x_p, W1)


def _tc_scale(t1, p0, p1):
    def body(t_ref, p0_ref, p1_ref, g_ref, dv_ref):
        deg = p0_ref[...] + p1_ref[...] + 1.0
        dv = lax.rsqrt(deg)
        dv_ref[...] = dv
        g_ref[...] = t_ref[...] * dv[:, 0:1]

    return pl.pallas_call(
        body,
        grid=(GRID,),
        in_specs=[
            pl.BlockSpec((RB, H1), lambda i: (i, 0)),
            pl.BlockSpec((RB, DEG_D), lambda i: (i, 0)),
            pl.BlockSpec((RB, DEG_D), lambda i: (i, 0)),
        ],
        out_specs=[
            pl.BlockSpec((RB, H1), lambda i: (i, 0)),
            pl.BlockSpec((RB, DEG_D), lambda i: (i, 0)),
        ],
        out_shape=[
            jax.ShapeDtypeStruct((NACC, H1), jnp.float32),
            jax.ShapeDtypeStruct((NACC, DEG_D), jnp.float32),
        ],
    )(t1, p0, p1)


def _tc_mid(sa, sb, g, dv, b, W, d_in, d_out):
    def body(sa_ref, sb_ref, g_ref, dv_ref, b_ref, w_ref, o_ref):
        dvc = dv_ref[...][:, 0:1]
        z = jnp.maximum(
            dvc * (sa_ref[...] + sb_ref[...] + g_ref[...]) + b_ref[...], 0.0)
        o_ref[...] = jnp.dot(z, w_ref[...],
                             preferred_element_type=jnp.float32) * dvc

    return pl.pallas_call(
        body,
        grid=(GRID,),
        in_specs=[
            pl.BlockSpec((RB, d_in), lambda i: (i, 0)),
            pl.BlockSpec((RB, d_in), lambda i: (i, 0)),
            pl.BlockSpec((RB, d_in), lambda i: (i, 0)),
            pl.BlockSpec((RB, DEG_D), lambda i: (i, 0)),
            pl.BlockSpec((1, d_in), lambda i: (0, 0)),
            pl.BlockSpec((d_in, d_out), lambda i: (0, 0)),
        ],
        out_specs=pl.BlockSpec((RB, d_out), lambda i: (i, 0)),
        out_shape=jax.ShapeDtypeStruct((NACC, d_out), jnp.float32),
    )(sa, sb, g, dv, b, W)


def _tc_final(sa, sb, g, dv, b):
    def body(sa_ref, sb_ref, g_ref, dv_ref, b_ref, o_ref):
        dvc = dv_ref[...][:, 0:1]
        z = dvc * (sa_ref[...] + sb_ref[...] + g_ref[...]) + b_ref[...]
        m = jnp.max(z, axis=1, keepdims=True)
        e = jnp.exp(z - m)
        o_ref[...] = z - m - jnp.log(jnp.sum(e, axis=1, keepdims=True))

    return pl.pallas_call(
        body,
        grid=(GRID,),
        in_specs=[
            pl.BlockSpec((RB, OUT), lambda i: (i, 0)),
            pl.BlockSpec((RB, OUT), lambda i: (i, 0)),
            pl.BlockSpec((RB, OUT), lambda i: (i, 0)),
            pl.BlockSpec((RB, DEG_D), lambda i: (i, 0)),
            pl.BlockSpec((1, OUT), lambda i: (0, 0)),
        ],
        out_specs=pl.BlockSpec((RB, OUT), lambda i: (i, 0)),
        out_shape=jax.ShapeDtypeStruct((NACC, OUT), jnp.float32),
    )(sa, sb, g, dv, b)


def kernel(x, edge_index, W1, b1, W2, b2, W3, b3):
    src = edge_index[0]
    dst = edge_index[1]
    pad = EP - E
    pad_idx = (jnp.arange(pad, dtype=jnp.int32) % (NACC - N - 16)) + N + 16
    src_f = jnp.concatenate([src, pad_idx])
    dst_f = jnp.concatenate([dst, pad_idx])
    src_p = src_f.reshape(TCH, K)
    dst_p = dst_f.reshape(TCH, K)
    src_w = src_f.reshape(EP // K64, K64)
    dst_w = dst_f.reshape(EP // K64, K64)
    ones16 = jnp.ones((K, DEG_D), jnp.float32)
    z16 = jnp.zeros((RPT, DEG_D), jnp.float32)
    z128 = jnp.zeros((RPT, 128), jnp.float32)
    z64 = jnp.zeros((RPT, OUT), jnp.float32)
    x_p = jnp.pad(x, ((0, NACC - N), (0, 0)))

    degP = _sc_deg(dst_p, ones16, z16)
    t1 = _tc_matmul(x_p, W1)
    g1, dv = _tc_scale(t1, degP[0], degP[1])
    S1 = _sc_agg128(g1, src_p, dst_p, z128)
    g2 = _tc_mid(S1[0], S1[1], g1, dv, b1.reshape(1, -1), W2, H1, H2)
    S2 = _sc_agg128(g2, src_p, dst_p, z128)
    g3 = _tc_mid(S2[0], S2[1], g2, dv, b2.reshape(1, -1), W3, H2, OUT)
    S3 = _sc_agg64(g3, src_w, dst_w, z64)
    out = _tc_final(S3[0], S3[1], g3, dv, b3.reshape(1, -1))
    return out[:N]
